# baseline (speedup 1.0000x reference)
import functools

import jax
import jax.numpy as jnp
from jax import lax
from jax.experimental import pallas as pl
from jax.experimental.pallas import tpu as pltpu
from jax.experimental.pallas import tpu_sc as plsc

N = 16384
D = 128
K = 25
KPAD = 128

NC, NS = 2, 16
NW = NC * NS
R = N // NW
CHUNK = 128
NCH = R // CHUNK


def _sc_gather_body(weight_h, bias_h, uni_h, target_h, noisep_h,
                    wt_o, bt_o, pnt_o, wn_o, bn_o, un_o,
                    idx_v, rows_v, f1_v, f2_v, sem):
    c = lax.axis_index("c")
    s = lax.axis_index("s")
    wid = s * NC + c
    base = wid * R

    for ch in range(NCH):
        pltpu.sync_copy(target_h.at[pl.ds(base + ch * CHUNK, CHUNK)],
                        idx_v.at[ch])

    for ch in range(NCH):
        idx = idx_v.at[ch]
        off = base + ch * CHUNK
        pltpu.async_copy(weight_h.at[idx], rows_v.at[ch], sem).wait()
        pltpu.sync_copy(rows_v.at[ch], wt_o.at[pl.ds(off, CHUNK)])
        pltpu.async_copy(bias_h.at[idx], f1_v.at[ch], sem).wait()
        pltpu.sync_copy(f1_v.at[ch], bt_o.at[pl.ds(off, CHUNK)])
        pltpu.async_copy(uni_h.at[idx], f2_v.at[ch], sem).wait()
        pltpu.sync_copy(f2_v.at[ch], pnt_o.at[pl.ds(off, CHUNK)])

    @pl.when(wid == 0)
    def _():
        pltpu.sync_copy(noisep_h, idx_v.at[0])
        idx = idx_v.at[0]
        pltpu.async_copy(weight_h.at[idx], rows_v.at[0], sem).wait()
        pltpu.sync_copy(rows_v.at[0], wn_o)
        pltpu.async_copy(bias_h.at[idx], f1_v.at[0], sem).wait()
        pltpu.sync_copy(f1_v.at[0], bn_o)
        pltpu.async_copy(uni_h.at[idx], f2_v.at[0], sem).wait()
        pltpu.sync_copy(f2_v.at[0], un_o)


_sc_gather = pl.kernel(
    _sc_gather_body,
    out_type=[
        jax.ShapeDtypeStruct((N, D), jnp.float32),
        jax.ShapeDtypeStruct((N,), jnp.float32),
        jax.ShapeDtypeStruct((N,), jnp.float32),
        jax.ShapeDtypeStruct((KPAD, D), jnp.float32),
        jax.ShapeDtypeStruct((KPAD,), jnp.float32),
        jax.ShapeDtypeStruct((KPAD,), jnp.float32),
    ],
    mesh=plsc.VectorSubcoreMesh(core_axis_name="c", subcore_axis_name="s",
                                num_cores=NC, num_subcores=NS),
    scratch_types=[
        pltpu.VMEM((NCH, CHUNK), jnp.int32),
        pltpu.VMEM((NCH, CHUNK, D), jnp.float32),
        pltpu.VMEM((NCH, CHUNK), jnp.float32),
        pltpu.VMEM((NCH, CHUNK), jnp.float32),
        pltpu.SemaphoreType.DMA,
    ],
)


BLK = 2048


def _tc_body(x_ref, wt_ref, bt_ref, wn_ref, bn_ref, un_ref,
             pmt_ref, pmn_ref, pnn_ref):
    x = x_ref[...]
    wt = wt_ref[...]
    pmt_ref[...] = jnp.exp(jnp.sum(x * wt, axis=1, keepdims=True)
                           + bt_ref[...])
    z = lax.dot_general(x, wn_ref[...], (((1,), (1,)), ((), ())),
                        preferred_element_type=jnp.float32)
    pmn_ref[...] = jnp.exp(z[:, :K] + bn_ref[0, :K][None, :])
    pnn_ref[...] = jnp.broadcast_to(un_ref[0, :K][None, :], (BLK, K))


_tc_dense = pl.pallas_call(
    _tc_body,
    grid=(N // BLK,),
    in_specs=[
        pl.BlockSpec((BLK, D), lambda i: (i, 0)),
        pl.BlockSpec((BLK, D), lambda i: (i, 0)),
        pl.BlockSpec((BLK, 1), lambda i: (i, 0)),
        pl.BlockSpec((KPAD, D), lambda i: (0, 0)),
        pl.BlockSpec((1, KPAD), lambda i: (0, 0)),
        pl.BlockSpec((1, KPAD), lambda i: (0, 0)),
    ],
    out_specs=[
        pl.BlockSpec((BLK, 1), lambda i: (i, 0)),
        pl.BlockSpec((BLK, K), lambda i: (i, 0)),
        pl.BlockSpec((BLK, K), lambda i: (i, 0)),
    ],
    out_shape=[
        jax.ShapeDtypeStruct((N, 1), jnp.float32),
        jax.ShapeDtypeStruct((N, K), jnp.float32),
        jax.ShapeDtypeStruct((N, K), jnp.float32),
    ],
)


def kernel(input, target, noise, weight, bias, unigram_prob):
    target = target.astype(jnp.int32)
    noise_pad = jnp.zeros((KPAD,), jnp.int32).at[:K].set(
        noise.astype(jnp.int32))
    wt_g, bt_g, pnt, wn_g, bn_g, un_g = _sc_gather(
        weight, bias, unigram_prob, target, noise_pad)
    pmt2, pmn, pnn = _tc_dense(
        input, wt_g, bt_g.reshape(N, 1), wn_g,
        bn_g.reshape(1, KPAD), un_g.reshape(1, KPAD))
    return (pmt2.reshape(N), pnt, pmn, pnn)


# speedup vs baseline: 1.1511x; 1.1511x over previous
import functools

import jax
import jax.numpy as jnp
from jax import lax
from jax.experimental import pallas as pl
from jax.experimental.pallas import tpu as pltpu
from jax.experimental.pallas import tpu_sc as plsc

N = 16384
D = 128
K = 25
KPAD = 128

NC, NS = 2, 16
NW = NC * NS
R = N // NW
CHUNK = 128
NCH = R // CHUNK


def _sc_gather_body(weight_h, bias_h, uni_h, target_h, noisep_h,
                    wt_o, bt_o, pnt_o, wn_o, bn_o, un_o,
                    idx_v, rows_v, f1_v, f2_v,
                    idxn_v, rowsn_v, f1n_v, f2n_v, gsem, wsem, nsem):
    c = lax.axis_index("c")
    s = lax.axis_index("s")
    wid = s * NC + c
    base = wid * R

    for ch in range(NCH):
        pltpu.sync_copy(target_h.at[pl.ds(base + ch * CHUNK, CHUNK)],
                        idx_v.at[ch])

    gathers = []
    for ch in range(NCH):
        idx = idx_v.at[ch]
        gathers.append(pltpu.async_copy(weight_h.at[idx], rows_v.at[ch], gsem))
        gathers.append(pltpu.async_copy(bias_h.at[idx], f1_v.at[ch], gsem))
        gathers.append(pltpu.async_copy(uni_h.at[idx], f2_v.at[ch], gsem))

    @pl.when(wid == 0)
    def _():
        pltpu.sync_copy(noisep_h, idxn_v)
        n1 = pltpu.async_copy(weight_h.at[idxn_v], rowsn_v, nsem)
        n2 = pltpu.async_copy(bias_h.at[idxn_v], f1n_v, nsem)
        n3 = pltpu.async_copy(uni_h.at[idxn_v], f2n_v, nsem)
        n1.wait()
        n2.wait()
        n3.wait()
        w1 = pltpu.async_copy(rowsn_v, wn_o, wsem)
        w2 = pltpu.async_copy(f1n_v, bn_o, wsem)
        w3 = pltpu.async_copy(f2n_v, un_o, wsem)
        w1.wait()
        w2.wait()
        w3.wait()

    for g in gathers:
        g.wait()

    writes = []
    for ch in range(NCH):
        off = base + ch * CHUNK
        writes.append(pltpu.async_copy(rows_v.at[ch],
                                       wt_o.at[pl.ds(off, CHUNK)], wsem))
        writes.append(pltpu.async_copy(f1_v.at[ch],
                                       bt_o.at[pl.ds(off, CHUNK)], wsem))
        writes.append(pltpu.async_copy(f2_v.at[ch],
                                       pnt_o.at[pl.ds(off, CHUNK)], wsem))
    for w in writes:
        w.wait()


_sc_gather = pl.kernel(
    _sc_gather_body,
    out_type=[
        jax.ShapeDtypeStruct((N, D), jnp.float32),
        jax.ShapeDtypeStruct((N,), jnp.float32),
        jax.ShapeDtypeStruct((N,), jnp.float32),
        jax.ShapeDtypeStruct((KPAD, D), jnp.float32),
        jax.ShapeDtypeStruct((KPAD,), jnp.float32),
        jax.ShapeDtypeStruct((KPAD,), jnp.float32),
    ],
    mesh=plsc.VectorSubcoreMesh(core_axis_name="c", subcore_axis_name="s",
                                num_cores=NC, num_subcores=NS),
    scratch_types=[
        pltpu.VMEM((NCH, CHUNK), jnp.int32),
        pltpu.VMEM((NCH, CHUNK, D), jnp.float32),
        pltpu.VMEM((NCH, CHUNK), jnp.float32),
        pltpu.VMEM((NCH, CHUNK), jnp.float32),
        pltpu.VMEM((KPAD,), jnp.int32),
        pltpu.VMEM((KPAD, D), jnp.float32),
        pltpu.VMEM((KPAD,), jnp.float32),
        pltpu.VMEM((KPAD,), jnp.float32),
        pltpu.SemaphoreType.DMA,
        pltpu.SemaphoreType.DMA,
        pltpu.SemaphoreType.DMA,
    ],
)


BLK = 2048


def _tc_body(x_ref, wt_ref, bt_ref, wn_ref, bn_ref, un_ref,
             pmt_ref, pmn_ref, pnn_ref):
    x = x_ref[...]
    wt = wt_ref[...]
    pmt_ref[...] = jnp.exp(jnp.sum(x * wt, axis=1, keepdims=True)
                           + bt_ref[...])
    z = lax.dot_general(x, wn_ref[...], (((1,), (1,)), ((), ())),
                        preferred_element_type=jnp.float32)
    pmn_ref[...] = jnp.exp(z[:, :K] + bn_ref[0, :K][None, :])
    pnn_ref[...] = jnp.broadcast_to(un_ref[0, :K][None, :], (BLK, K))


_tc_dense = pl.pallas_call(
    _tc_body,
    grid=(N // BLK,),
    in_specs=[
        pl.BlockSpec((BLK, D), lambda i: (i, 0)),
        pl.BlockSpec((BLK, D), lambda i: (i, 0)),
        pl.BlockSpec((BLK, 1), lambda i: (i, 0)),
        pl.BlockSpec((KPAD, D), lambda i: (0, 0)),
        pl.BlockSpec((1, KPAD), lambda i: (0, 0)),
        pl.BlockSpec((1, KPAD), lambda i: (0, 0)),
    ],
    out_specs=[
        pl.BlockSpec((BLK, 1), lambda i: (i, 0)),
        pl.BlockSpec((BLK, K), lambda i: (i, 0)),
        pl.BlockSpec((BLK, K), lambda i: (i, 0)),
    ],
    out_shape=[
        jax.ShapeDtypeStruct((N, 1), jnp.float32),
        jax.ShapeDtypeStruct((N, K), jnp.float32),
        jax.ShapeDtypeStruct((N, K), jnp.float32),
    ],
)


def kernel(input, target, noise, weight, bias, unigram_prob):
    target = target.astype(jnp.int32)
    noise_pad = jnp.zeros((KPAD,), jnp.int32).at[:K].set(
        noise.astype(jnp.int32))
    wt_g, bt_g, pnt, wn_g, bn_g, un_g = _sc_gather(
        weight, bias, unigram_prob, target, noise_pad)
    pmt2, pmn, pnn = _tc_dense(
        input, wt_g, bt_g.reshape(N, 1), wn_g,
        bn_g.reshape(1, KPAD), un_g.reshape(1, KPAD))
    return (pmt2.reshape(N), pnt, pmn, pnn)


# speedup vs baseline: 1.7338x; 1.5062x over previous
import jax
import jax.numpy as jnp
from jax import lax
from jax.experimental import pallas as pl
from jax.experimental.pallas import tpu as pltpu
from jax.experimental.pallas import tpu_sc as plsc

N = 16384
D = 128
K = 25
KPAD = 32

NC, NS = 2, 16
NW = NC * NS
R = N // NW
CHUNK = 128
NCH = R // CHUNK
GRP = CHUNK // 16


_DNUMS = lax.GatherDimensionNumbers(offset_dims=(), collapsed_slice_dims=(0,),
                                    start_index_map=(0,))


def _hsum_all_lanes(a, lane):
    for sh in (8, 4, 2, 1):
        idx = (lane ^ sh)[:, None]
        a = a + lax.gather(a, idx, _DNUMS, slice_sizes=(1,),
                           mode=lax.GatherScatterMode.PROMISE_IN_BOUNDS)
    return a


def _compute_chunk(rows_v, xin_v, dots_v, slot, ch):
    lane = lax.iota(jnp.int32, 16)

    def group_body(g, carry):
        rowbase = g * 16
        tot = jnp.zeros((16,), jnp.float32)
        for r in range(16):
            row = rowbase + r
            acc = (rows_v[slot, row, pl.ds(0, 16)]
                   * xin_v[slot, row, pl.ds(0, 16)])
            for cc in range(1, 8):
                acc = acc + (rows_v[slot, row, pl.ds(cc * 16, 16)]
                             * xin_v[slot, row, pl.ds(cc * 16, 16)])
            tot = jnp.where(lane == r, _hsum_all_lanes(acc, lane), tot)
        dots_v[pl.ds(ch * CHUNK + rowbase, 16)] = tot
        return carry

    lax.fori_loop(0, GRP, group_body, 0)


def _sc_main_body(weight_h, bias_h, uni_h, target_h, input_h,
                  pmt_o, pnt_o,
                  idx_v, rows_v, xin_v, f1_v, f2_v, dots_v, pm_v,
                  gsem0, gsem1, fsem, wsem):
    c = lax.axis_index("c")
    s = lax.axis_index("s")
    wid = s * NC + c
    base = wid * R

    for ch in range(NCH):
        pltpu.sync_copy(target_h.at[pl.ds(base + ch * CHUNK, CHUNK)],
                        idx_v.at[ch])

    fcopies = []
    for ch in range(NCH):
        sl = pl.ds(ch * CHUNK, CHUNK)
        fcopies.append(pltpu.async_copy(bias_h.at[idx_v.at[ch]],
                                        f1_v.at[sl], fsem))
        fcopies.append(pltpu.async_copy(uni_h.at[idx_v.at[ch]],
                                        f2_v.at[sl], fsem))

    sems = (gsem0, gsem1)

    def fire(ch):
        slot = ch % 2
        return (pltpu.async_copy(weight_h.at[idx_v.at[ch]],
                                 rows_v.at[slot], sems[slot]),
                pltpu.async_copy(input_h.at[pl.ds(base + ch * CHUNK, CHUNK)],
                                 xin_v.at[slot], sems[slot]))

    pend = fire(0)
    for ch in range(NCH):
        nxt = fire(ch + 1) if ch + 1 < NCH else None
        pend[0].wait()
        pend[1].wait()
        _compute_chunk(rows_v, xin_v, dots_v, ch % 2, ch)
        pend = nxt

    for f in fcopies:
        f.wait()

    for g in range(R // 16):
        sl = pl.ds(g * 16, 16)
        pm_v[sl] = jnp.exp(dots_v[sl] + f1_v[sl])
    w1 = pltpu.async_copy(pm_v, pmt_o.at[pl.ds(base, R)], wsem)
    w2 = pltpu.async_copy(f2_v, pnt_o.at[pl.ds(base, R)], wsem)
    w1.wait()
    w2.wait()


_sc_main = pl.kernel(
    _sc_main_body,
    out_type=[
        jax.ShapeDtypeStruct((N,), jnp.float32),
        jax.ShapeDtypeStruct((N,), jnp.float32),
    ],
    mesh=plsc.VectorSubcoreMesh(core_axis_name="c", subcore_axis_name="s",
                                num_cores=NC, num_subcores=NS),
    scratch_types=[
        pltpu.VMEM((NCH, CHUNK), jnp.int32),
        pltpu.VMEM((2, CHUNK, D), jnp.float32),
        pltpu.VMEM((2, CHUNK, D), jnp.float32),
        pltpu.VMEM((R,), jnp.float32),
        pltpu.VMEM((R,), jnp.float32),
        pltpu.VMEM((R,), jnp.float32),
        pltpu.VMEM((R,), jnp.float32),
        pltpu.SemaphoreType.DMA,
        pltpu.SemaphoreType.DMA,
        pltpu.SemaphoreType.DMA,
        pltpu.SemaphoreType.DMA,
    ],
)


BLK = 2048


def _tc_body(noise_sref, x_ref, w_any, b_any, u_any,
             pmn_ref, pnn_ref, wn_v, b8_v, u8_v, bnun_v, sem):
    @pl.when(pl.program_id(0) == 0)
    def _():
        cps = []
        for k in range(K):
            idx = noise_sref[k]
            base = pl.multiple_of((idx // 128) * 128, 128)
            cps.append(pltpu.make_async_copy(
                w_any.at[pl.ds(idx, 1), :], wn_v.at[pl.ds(k, 1), :], sem))
            cps.append(pltpu.make_async_copy(
                b_any.at[pl.ds(base, 128)], b8_v.at[k], sem))
            cps.append(pltpu.make_async_copy(
                u_any.at[pl.ds(base, 128)], u8_v.at[k], sem))
        for cp in cps:
            cp.start()
        for cp in cps:
            cp.wait()
        lane128 = lax.iota(jnp.int32, 128)
        lanek = lax.iota(jnp.int32, KPAD)
        bn_acc = jnp.zeros((KPAD,), jnp.float32)
        un_acc = jnp.zeros((KPAD,), jnp.float32)
        for k in range(K):
            col = noise_sref[k] % 128
            bval = jnp.sum(jnp.where(lane128 == col, b8_v[k], 0.0))
            uval = jnp.sum(jnp.where(lane128 == col, u8_v[k], 0.0))
            bn_acc = jnp.where(lanek == k, bval, bn_acc)
            un_acc = jnp.where(lanek == k, uval, un_acc)
        bnun_v[0] = bn_acc
        bnun_v[1] = un_acc

    x = x_ref[...]
    z = lax.dot_general(x, wn_v[...], (((1,), (1,)), ((), ())),
                        preferred_element_type=jnp.float32)
    pmn_ref[...] = jnp.exp(z[:, :K] + bnun_v[0][:K][None, :])
    pnn_ref[...] = jnp.broadcast_to(bnun_v[1][:K][None, :], (BLK, K))


_tc_dense = pl.pallas_call(
    _tc_body,
    grid=(N // BLK,),
    in_specs=[
        pl.BlockSpec(memory_space=pltpu.SMEM),
        pl.BlockSpec((BLK, D), lambda i: (i, 0)),
        pl.BlockSpec(memory_space=pl.ANY),
        pl.BlockSpec(memory_space=pl.ANY),
        pl.BlockSpec(memory_space=pl.ANY),
    ],
    out_specs=[
        pl.BlockSpec((BLK, K), lambda i: (i, 0)),
        pl.BlockSpec((BLK, K), lambda i: (i, 0)),
    ],
    out_shape=[
        jax.ShapeDtypeStruct((N, K), jnp.float32),
        jax.ShapeDtypeStruct((N, K), jnp.float32),
    ],
    scratch_shapes=[
        pltpu.VMEM((KPAD, D), jnp.float32),
        pltpu.VMEM((K, 128), jnp.float32),
        pltpu.VMEM((K, 128), jnp.float32),
        pltpu.VMEM((2, KPAD), jnp.float32),
        pltpu.SemaphoreType.DMA,
    ],
)


def kernel(input, target, noise, weight, bias, unigram_prob):
    target = target.astype(jnp.int32)
    noise = noise.astype(jnp.int32)
    pmt, pnt = _sc_main(weight, bias, unigram_prob, target, input)
    pad = (-bias.shape[0]) % 128
    bias_p = jnp.pad(bias, (0, pad))
    uni_p = jnp.pad(unigram_prob, (0, pad))
    pmn, pnn = _tc_dense(noise, input, weight, bias_p, uni_p)
    return (pmt, pnt, pmn, pnn)
